# Initial kernel scaffold; baseline (speedup 1.0000x reference)
import jax
import jax.numpy as jnp
from jax.experimental import pallas as pl


def kernel(x, s_ei, t_ei, Wp, bp, g1_W, g1_b, bn1_g, bn1_b, r1_W, r1_b, g2_W, g2_b, bn2_g, bn2_b, g3_W, g3_b, bn3_g, bn3_b, r3_W, r3_b, g4_W, g4_b, bn4_g, bn4_b, a1_W, a1_b, a2_W, a2_b, ln_g, ln_b, c1_W, c1_b, c2_W, c2_b):
    raise NotImplementedError("write your pallas kernel here")


# speedup vs baseline: 2.5018x; 2.5018x over previous
import functools
import math

import jax
import jax.numpy as jnp
from jax.experimental import pallas as pl

B, T, N, C = 16, 50, 55, 6
M = B * T * N
NT = T * N
RT = 1000
S_BN = 1.0 / math.sqrt(1.0 + 1e-5)

_f32 = jnp.float32


def _row(i):
    return (i, 0)


def _rep(i):
    return (0, 0)


def _gelu(v):
    return 0.5 * v * (1.0 + jax.lax.erf(v * (1.0 / math.sqrt(2.0))))


def _ka_body(x_ref, Wp_ref, bp_ref, g1W_ref, r1W_ref, r1b_ref, dis_ref,
             up1_ref, res1_ref):
    h0 = jnp.dot(x_ref[...], Wp_ref[...],
                 preferred_element_type=_f32) + bp_ref[...]
    up1_ref[...] = jnp.dot(h0, g1W_ref[...],
                           preferred_element_type=_f32) * dis_ref[...]
    res1_ref[...] = jnp.dot(h0, r1W_ref[...],
                            preferred_element_type=_f32) + r1b_ref[...]


def _ka(x, Wp, bp, g1W, r1W, r1b, dis_s_col):
    return pl.pallas_call(
        _ka_body,
        grid=(M // RT,),
        in_specs=[
            pl.BlockSpec((RT, 8), _row),
            pl.BlockSpec((8, 64), _rep),
            pl.BlockSpec((1, 64), _rep),
            pl.BlockSpec((64, 128), _rep),
            pl.BlockSpec((64, 128), _rep),
            pl.BlockSpec((1, 128), _rep),
            pl.BlockSpec((RT, 1), _row),
        ],
        out_specs=[pl.BlockSpec((RT, 128), _row),
                   pl.BlockSpec((RT, 128), _row)],
        out_shape=[jax.ShapeDtypeStruct((M, 128), _f32),
                   jax.ShapeDtypeStruct((M, 128), _f32)],
    )(x, Wp, bp, g1W, r1W, r1b, dis_s_col)


def _post_pre_body(acc_ref, up_ref, res_ref, disa_ref, gb_ref, bng_ref,
                   bnb_ref, W_ref, disb_ref, h_ref, upn_ref):
    v = disa_ref[...] * (acc_ref[...] + up_ref[...]) + gb_ref[...]
    h = _gelu(v * bng_ref[...] + bnb_ref[...] + res_ref[...])
    h_ref[...] = h
    upn_ref[...] = jnp.dot(h, W_ref[...],
                           preferred_element_type=_f32) * disb_ref[...]


def _post_pre(acc, up, res, dis_a, gb, bng_eff, bnb, W, dis_b):
    Fin = up.shape[1]
    Fout = W.shape[1]
    return pl.pallas_call(
        _post_pre_body,
        grid=(M // RT,),
        in_specs=[
            pl.BlockSpec((RT, Fin), _row),
            pl.BlockSpec((RT, Fin), _row),
            pl.BlockSpec((RT, Fin), _row),
            pl.BlockSpec((RT, 1), _row),
            pl.BlockSpec((1, Fin), _rep),
            pl.BlockSpec((1, Fin), _rep),
            pl.BlockSpec((1, Fin), _rep),
            pl.BlockSpec((Fin, Fout), _rep),
            pl.BlockSpec((RT, 1), _row),
        ],
        out_specs=[pl.BlockSpec((RT, Fin), _row),
                   pl.BlockSpec((RT, Fout), _row)],
        out_shape=[jax.ShapeDtypeStruct((M, Fin), _f32),
                   jax.ShapeDtypeStruct((M, Fout), _f32)],
    )(acc, up, res, dis_a, gb, bng_eff, bnb, W, dis_b)


def _post_pre2_body(acc_ref, up_ref, res_ref, disa_ref, gb_ref, bng_ref,
                    bnb_ref, W_ref, rW_ref, rb_ref, disb_ref,
                    upn_ref, resn_ref):
    v = disa_ref[...] * (acc_ref[...] + up_ref[...]) + gb_ref[...]
    h = _gelu(v * bng_ref[...] + bnb_ref[...] + res_ref[...])
    upn_ref[...] = jnp.dot(h, W_ref[...],
                           preferred_element_type=_f32) * disb_ref[...]
    resn_ref[...] = jnp.dot(h, rW_ref[...],
                            preferred_element_type=_f32) + rb_ref[...]


def _post_pre2(acc, up, res, dis_a, gb, bng_eff, bnb, W, rW, rb, dis_b):
    Fin = up.shape[1]
    Fout = W.shape[1]
    return pl.pallas_call(
        _post_pre2_body,
        grid=(M // RT,),
        in_specs=[
            pl.BlockSpec((RT, Fin), _row),
            pl.BlockSpec((RT, Fin), _row),
            pl.BlockSpec((RT, Fin), _row),
            pl.BlockSpec((RT, 1), _row),
            pl.BlockSpec((1, Fin), _rep),
            pl.BlockSpec((1, Fin), _rep),
            pl.BlockSpec((1, Fin), _rep),
            pl.BlockSpec((Fin, Fout), _rep),
            pl.BlockSpec((Fin, Fout), _rep),
            pl.BlockSpec((1, Fout), _rep),
            pl.BlockSpec((RT, 1), _row),
        ],
        out_specs=[pl.BlockSpec((RT, Fout), _row),
                   pl.BlockSpec((RT, Fout), _row)],
        out_shape=[jax.ShapeDtypeStruct((M, Fout), _f32),
                   jax.ShapeDtypeStruct((M, Fout), _f32)],
    )(acc, up, res, dis_a, gb, bng_eff, bnb, W, rW, rb, dis_b)


_RG = 8 * N


def _kpool_body(acc_ref, up_ref, res_ref, dis_ref, gb_ref, bng_ref, bnb_ref,
                out_ref):
    v = dis_ref[...] * (acc_ref[...] + up_ref[...]) + gb_ref[...]
    h4 = _gelu(v * bng_ref[...] + bnb_ref[...] + res_ref[...])
    gi = jax.lax.broadcasted_iota(jnp.int32, (8, _RG), 0)
    ri = jax.lax.broadcasted_iota(jnp.int32, (8, _RG), 1)
    sel = jnp.where(ri // N == gi, 1.0 / N, 0.0).astype(_f32)
    out_ref[...] = jnp.dot(sel, h4, preferred_element_type=_f32)


def _kpool(acc, up, res, dis_t_col, gb, bng_eff, bnb):
    return pl.pallas_call(
        _kpool_body,
        grid=(M // _RG,),
        in_specs=[
            pl.BlockSpec((_RG, 256), _row),
            pl.BlockSpec((_RG, 256), _row),
            pl.BlockSpec((_RG, 256), _row),
            pl.BlockSpec((_RG, 1), _row),
            pl.BlockSpec((1, 256), _rep),
            pl.BlockSpec((1, 256), _rep),
            pl.BlockSpec((1, 256), _rep),
        ],
        out_specs=pl.BlockSpec((8, 256), _row),
        out_shape=jax.ShapeDtypeStruct((B * T, 256), _f32),
    )(acc, up, res, dis_t_col, gb, bng_eff, bnb)


def _khead_body(hT_ref, a1_ref, a1b_ref, a2_ref, lng_ref, lnb_ref,
                c1_ref, c1b_ref, c2_ref, c2b_ref, out_ref):
    ti = jax.lax.broadcasted_iota(jnp.int32, (56, 1), 0)
    tmask = ti < T
    for b in range(B):
        x = hT_ref[b]
        t = jnp.tanh(jnp.dot(x, a1_ref[...],
                             preferred_element_type=_f32) + a1b_ref[...])
        logits = jnp.dot(t, a2_ref[...], preferred_element_type=_f32)
        logits = jnp.where(tmask, logits, -1e30)
        e = jnp.exp(logits - jnp.max(logits, axis=0, keepdims=True))
        e = jnp.where(tmask, e, 0.0)
        w = e / jnp.sum(e, axis=0, keepdims=True)
        pooled = jnp.sum(x * w, axis=0, keepdims=True)
        mu = jnp.mean(pooled, axis=1, keepdims=True)
        var = jnp.mean((pooled - mu) ** 2, axis=1, keepdims=True)
        z = (pooled - mu) * jax.lax.rsqrt(var + 1e-5) * lng_ref[...] \
            + lnb_ref[...]
        z1 = _gelu(jnp.dot(z, c1_ref[...],
                           preferred_element_type=_f32) + c1b_ref[...])
        out_ref[pl.ds(b, 1), :] = jnp.dot(
            z1, c2_ref[...], preferred_element_type=_f32) + c2b_ref[...]


def _khead(hT, a1, a1b, a2, a2b, lng, lnb, c1, c1b, c2, c2b):
    del a2b
    return pl.pallas_call(
        _khead_body,
        grid=(1,),
        in_specs=[
            pl.BlockSpec((B, 56, 256), lambda i: (0, 0, 0)),
            pl.BlockSpec((256, 64), _rep),
            pl.BlockSpec((1, 64), _rep),
            pl.BlockSpec((64, 1), _rep),
            pl.BlockSpec((1, 256), _rep),
            pl.BlockSpec((1, 256), _rep),
            pl.BlockSpec((256, 256), _rep),
            pl.BlockSpec((1, 256), _rep),
            pl.BlockSpec((256, 104), _rep),
            pl.BlockSpec((1, 104), _rep),
        ],
        out_specs=pl.BlockSpec((B, 104), _rep),
        out_shape=jax.ShapeDtypeStruct((B, 104), _f32),
    )(hT, a1, a1b, a2, lng, lnb, c1, c1b, c2, c2b)


def _batch_edges(ei, reps, n):
    offs = jnp.arange(reps, dtype=ei.dtype) * n
    r = (ei[0][None, :] + offs[:, None]).reshape(-1)
    c = (ei[1][None, :] + offs[:, None]).reshape(-1)
    return r, c


def _scatter(up, rows, cols):
    return jnp.zeros_like(up).at[cols].add(up[rows])


def kernel(x, s_ei, t_ei, Wp, bp, g1_W, g1_b, bn1_g, bn1_b, r1_W, r1_b,
           g2_W, g2_b, bn2_g, bn2_b, g3_W, g3_b, bn3_g, bn3_b, r3_W, r3_b,
           g4_W, g4_b, bn4_g, bn4_b, a1_W, a1_b, a2_W, a2_b, ln_g, ln_b,
           c1_W, c1_b, c2_W, c2_b):
    xf = jnp.pad(x.reshape(M, C), ((0, 0), (0, 2)))

    deg_s = jnp.zeros((N,), _f32).at[s_ei[1]].add(1.0) + 1.0
    dis_s = jax.lax.rsqrt(deg_s)
    deg_t = jnp.zeros((NT,), _f32).at[t_ei[1]].add(1.0) + 1.0
    dis_t = jax.lax.rsqrt(deg_t)
    dis_s_col = jnp.tile(dis_s, B * T)[:, None]
    dis_t_col = jnp.tile(dis_t, B)[:, None]

    sr, sc = _batch_edges(s_ei, B * T, N)
    tr, tc = _batch_edges(t_ei, B, NT)

    def r2(v):
        return v[None, :]

    up1, res1 = _ka(xf, jnp.pad(Wp, ((0, 2), (0, 0))), r2(bp),
                    g1_W, r1_W, r2(r1_b), dis_s_col)
    acc1 = _scatter(up1, sr, sc)
    h1, up2 = _post_pre(acc1, up1, res1, dis_s_col, r2(g1_b),
                        r2(bn1_g) * S_BN, r2(bn1_b), g2_W, dis_s_col)
    acc2 = _scatter(up2, sr, sc)
    up3, res3 = _post_pre2(acc2, up2, h1, dis_s_col, r2(g2_b),
                           r2(bn2_g) * S_BN, r2(bn2_b), g3_W, r3_W,
                           r2(r3_b), dis_t_col)
    acc3 = _scatter(up3, tr, tc)
    h3, up4 = _post_pre(acc3, up3, res3, dis_t_col, r2(g3_b),
                        r2(bn3_g) * S_BN, r2(bn3_b), g4_W, dis_t_col)
    acc4 = _scatter(up4, tr, tc)
    pooled = _kpool(acc4, up4, h3, dis_t_col, r2(g4_b),
                    r2(bn4_g) * S_BN, r2(bn4_b))

    hT = jnp.pad(pooled.reshape(B, T, 256), ((0, 0), (0, 6), (0, 0)))
    out = _khead(hT, a1_W, r2(a1_b), a2_W, a2_b, r2(ln_g), r2(ln_b),
                 c1_W, r2(c1_b), jnp.pad(c2_W, ((0, 0), (0, 4))),
                 jnp.pad(r2(c2_b), ((0, 0), (0, 4))))
    return out[:, :100]


# speedup vs baseline: 2.8076x; 1.1222x over previous
import functools
import math

import jax
import jax.numpy as jnp
from jax import lax
from jax.experimental import pallas as pl
from jax.experimental.pallas import tpu as pltpu
from jax.experimental.pallas import tpu_sc as plsc

B, T, N, C = 16, 50, 55, 6
NT = T * N
NTP = NT + 2
MP = B * NTP
RT = 1024
S_BN = 1.0 / math.sqrt(1.0 + 1e-5)
NSUB = 16

_f32 = jnp.float32
_i32 = jnp.int32


def _row(i):
    return (i, 0)


def _rep(i):
    return (0, 0)


def _gelu(v):
    return 0.5 * v * (1.0 + jax.lax.erf(v * (1.0 / math.sqrt(2.0))))


def _ka_body(x_ref, Wp_ref, bp_ref, g1W_ref, r1W_ref, r1b_ref, dis_ref,
             up1_ref, res1_ref):
    h0 = jnp.dot(x_ref[...], Wp_ref[...],
                 preferred_element_type=_f32) + bp_ref[...]
    up1_ref[...] = jnp.dot(h0, g1W_ref[...],
                           preferred_element_type=_f32) * dis_ref[...]
    res1_ref[...] = jnp.dot(h0, r1W_ref[...],
                            preferred_element_type=_f32) + r1b_ref[...]


def _ka(x, Wp, bp, g1W, r1W, r1b, dis_s_col):
    return pl.pallas_call(
        _ka_body,
        grid=(MP // RT,),
        in_specs=[
            pl.BlockSpec((RT, 8), _row),
            pl.BlockSpec((8, 64), _rep),
            pl.BlockSpec((1, 64), _rep),
            pl.BlockSpec((64, 128), _rep),
            pl.BlockSpec((64, 128), _rep),
            pl.BlockSpec((1, 128), _rep),
            pl.BlockSpec((RT, 1), _row),
        ],
        out_specs=[pl.BlockSpec((RT, 128), _row),
                   pl.BlockSpec((RT, 128), _row)],
        out_shape=[jax.ShapeDtypeStruct((MP, 128), _f32),
                   jax.ShapeDtypeStruct((MP, 128), _f32)],
    )(x, Wp, bp, g1W, r1W, r1b, dis_s_col)


def _post_pre_body(acc_ref, up_ref, res_ref, disa_ref, gb_ref, bng_ref,
                   bnb_ref, W_ref, disb_ref, h_ref, upn_ref):
    v = disa_ref[...] * (acc_ref[...] + up_ref[...]) + gb_ref[...]
    h = _gelu(v * bng_ref[...] + bnb_ref[...] + res_ref[...])
    h_ref[...] = h
    upn_ref[...] = jnp.dot(h, W_ref[...],
                           preferred_element_type=_f32) * disb_ref[...]


def _post_pre(acc, up, res, dis_a, gb, bng_eff, bnb, W, dis_b):
    Fin = up.shape[1]
    Fout = W.shape[1]
    return pl.pallas_call(
        _post_pre_body,
        grid=(MP // RT,),
        in_specs=[
            pl.BlockSpec((RT, Fin), _row),
            pl.BlockSpec((RT, Fin), _row),
            pl.BlockSpec((RT, Fin), _row),
            pl.BlockSpec((RT, 1), _row),
            pl.BlockSpec((1, Fin), _rep),
            pl.BlockSpec((1, Fin), _rep),
            pl.BlockSpec((1, Fin), _rep),
            pl.BlockSpec((Fin, Fout), _rep),
            pl.BlockSpec((RT, 1), _row),
        ],
        out_specs=[pl.BlockSpec((RT, Fin), _row),
                   pl.BlockSpec((RT, Fout), _row)],
        out_shape=[jax.ShapeDtypeStruct((MP, Fin), _f32),
                   jax.ShapeDtypeStruct((MP, Fout), _f32)],
    )(acc, up, res, dis_a, gb, bng_eff, bnb, W, dis_b)


def _post_pre2_body(acc_ref, up_ref, res_ref, disa_ref, gb_ref, bng_ref,
                    bnb_ref, W_ref, rW_ref, rb_ref, disb_ref,
                    upn_ref, resn_ref):
    v = disa_ref[...] * (acc_ref[...] + up_ref[...]) + gb_ref[...]
    h = _gelu(v * bng_ref[...] + bnb_ref[...] + res_ref[...])
    upn_ref[...] = jnp.dot(h, W_ref[...],
                           preferred_element_type=_f32) * disb_ref[...]
    resn_ref[...] = jnp.dot(h, rW_ref[...],
                            preferred_element_type=_f32) + rb_ref[...]


def _post_pre2(acc, up, res, dis_a, gb, bng_eff, bnb, W, rW, rb, dis_b):
    Fin = up.shape[1]
    Fout = W.shape[1]
    return pl.pallas_call(
        _post_pre2_body,
        grid=(MP // RT,),
        in_specs=[
            pl.BlockSpec((RT, Fin), _row),
            pl.BlockSpec((RT, Fin), _row),
            pl.BlockSpec((RT, Fin), _row),
            pl.BlockSpec((RT, 1), _row),
            pl.BlockSpec((1, Fin), _rep),
            pl.BlockSpec((1, Fin), _rep),
            pl.BlockSpec((1, Fin), _rep),
            pl.BlockSpec((Fin, Fout), _rep),
            pl.BlockSpec((Fin, Fout), _rep),
            pl.BlockSpec((1, Fout), _rep),
            pl.BlockSpec((RT, 1), _row),
        ],
        out_specs=[pl.BlockSpec((RT, Fout), _row),
                   pl.BlockSpec((RT, Fout), _row)],
        out_shape=[jax.ShapeDtypeStruct((MP, Fout), _f32),
                   jax.ShapeDtypeStruct((MP, Fout), _f32)],
    )(acc, up, res, dis_a, gb, bng_eff, bnb, W, rW, rb, dis_b)


def _kpool_body(acc_ref, up_ref, res_ref, dis_ref, gb_ref, bng_ref, bnb_ref,
                out_ref):
    v = dis_ref[...] * (acc_ref[...] + up_ref[...]) + gb_ref[...]
    h4 = _gelu(v * bng_ref[...] + bnb_ref[...] + res_ref[...])
    ti = jax.lax.broadcasted_iota(_i32, (56, NTP), 0)
    ri = jax.lax.broadcasted_iota(_i32, (56, NTP), 1)
    sel = jnp.where((ri // N == ti) & (ri < NT), 1.0 / N, 0.0).astype(_f32)
    out_ref[0] = jnp.dot(sel, h4, preferred_element_type=_f32)


def _kpool(acc, up, res, dis_t_col, gb, bng_eff, bnb):
    return pl.pallas_call(
        _kpool_body,
        grid=(B,),
        in_specs=[
            pl.BlockSpec((NTP, 256), _row),
            pl.BlockSpec((NTP, 256), _row),
            pl.BlockSpec((NTP, 256), _row),
            pl.BlockSpec((NTP, 1), _row),
            pl.BlockSpec((1, 256), _rep),
            pl.BlockSpec((1, 256), _rep),
            pl.BlockSpec((1, 256), _rep),
        ],
        out_specs=pl.BlockSpec((1, 56, 256), lambda i: (i, 0, 0)),
        out_shape=jax.ShapeDtypeStruct((B, 56, 256), _f32),
    )(acc, up, res, dis_t_col, gb, bng_eff, bnb)


def _khead_body(hT_ref, a1_ref, a1b_ref, a2_ref, lng_ref, lnb_ref,
                c1_ref, c1b_ref, c2_ref, c2b_ref, out_ref):
    ti = jax.lax.broadcasted_iota(_i32, (56, 1), 0)
    tmask = ti < T
    for b in range(B):
        x = hT_ref[b]
        t = jnp.tanh(jnp.dot(x, a1_ref[...],
                             preferred_element_type=_f32) + a1b_ref[...])
        logits = jnp.dot(t, a2_ref[...], preferred_element_type=_f32)
        logits = jnp.where(tmask, logits, -1e30)
        e = jnp.exp(logits - jnp.max(logits, axis=0, keepdims=True))
        e = jnp.where(tmask, e, 0.0)
        w = e / jnp.sum(e, axis=0, keepdims=True)
        pooled = jnp.sum(x * w, axis=0, keepdims=True)
        mu = jnp.mean(pooled, axis=1, keepdims=True)
        var = jnp.mean((pooled - mu) ** 2, axis=1, keepdims=True)
        z = (pooled - mu) * jax.lax.rsqrt(var + 1e-5) * lng_ref[...] \
            + lnb_ref[...]
        z1 = _gelu(jnp.dot(z, c1_ref[...],
                           preferred_element_type=_f32) + c1b_ref[...])
        out_ref[pl.ds(b, 1), :] = jnp.dot(
            z1, c2_ref[...], preferred_element_type=_f32) + c2b_ref[...]


def _khead(hT, a1, a1b, a2, lng, lnb, c1, c1b, c2, c2b):
    return pl.pallas_call(
        _khead_body,
        grid=(1,),
        in_specs=[
            pl.BlockSpec((B, 56, 256), lambda i: (0, 0, 0)),
            pl.BlockSpec((256, 64), _rep),
            pl.BlockSpec((1, 64), _rep),
            pl.BlockSpec((64, 1), _rep),
            pl.BlockSpec((1, 256), _rep),
            pl.BlockSpec((1, 256), _rep),
            pl.BlockSpec((256, 256), _rep),
            pl.BlockSpec((1, 256), _rep),
            pl.BlockSpec((256, 104), _rep),
            pl.BlockSpec((1, 104), _rep),
        ],
        out_specs=pl.BlockSpec((B, 104), _rep),
        out_shape=jax.ShapeDtypeStruct((B, 104), _f32),
    )(hT, a1, a1b, a2, lng, lnb, c1, c1b, c2, c2b)


def _flat_edges(s_ei, t_ei):
    goffs = (jnp.arange(T, dtype=_i32) * N)[:, None]
    sr0 = (s_ei[0][None, :] + goffs).reshape(-1)
    sc0 = (s_ei[1][None, :] + goffs).reshape(-1)
    offs = jnp.arange(B, dtype=_i32) * NTP
    sri = (sr0[None, :] + offs[:, None]).reshape(-1)
    sci = (sc0[None, :] + offs[:, None]).reshape(-1)
    tri = (t_ei[0][None, :] + offs[:, None]).reshape(-1)
    tci = (t_ei[1][None, :] + offs[:, None]).reshape(-1)
    return sri, sci, tri, tci


def kernel(x, s_ei, t_ei, Wp, bp, g1_W, g1_b, bn1_g, bn1_b, r1_W, r1_b,
           g2_W, g2_b, bn2_g, bn2_b, g3_W, g3_b, bn3_g, bn3_b, r3_W, r3_b,
           g4_W, g4_b, bn4_g, bn4_b, a1_W, a1_b, a2_W, a2_b, ln_g, ln_b,
           c1_W, c1_b, c2_W, c2_b):
    xf = jnp.pad(x.reshape(B, NT, C), ((0, 0), (0, 2), (0, 2)))
    xf = xf.reshape(MP, 8)

    deg_s = jnp.zeros((N,), _f32).at[s_ei[1]].add(1.0)
    deg_t = jnp.zeros((NT,), _f32).at[t_ei[1]].add(1.0)
    dis_s = jax.lax.rsqrt(deg_s + 1.0)
    dis_t = jax.lax.rsqrt(deg_t + 1.0)
    pad1 = jnp.ones((2,), _f32)
    dis_s_col = jnp.tile(jnp.concatenate([jnp.tile(dis_s, T), pad1]),
                         B)[:, None]
    dis_t_col = jnp.tile(jnp.concatenate([dis_t, pad1]), B)[:, None]

    sri, sci, tri, tci = _flat_edges(s_ei, t_ei)

    def scat(up, r, c):
        return jnp.zeros(up.shape, _f32).at[c].add(up[r])

    def r2(v):
        return v[None, :]

    up1, res1 = _ka(xf, jnp.pad(Wp, ((0, 2), (0, 0))), r2(bp),
                    g1_W, r1_W, r2(r1_b), dis_s_col)
    acc1 = scat(up1, sri, sci)
    h1, up2 = _post_pre(acc1, up1, res1, dis_s_col, r2(g1_b),
                        r2(bn1_g) * S_BN, r2(bn1_b), g2_W, dis_s_col)
    acc2 = scat(up2, sri, sci)
    up3, res3 = _post_pre2(acc2, up2, h1, dis_s_col, r2(g2_b),
                           r2(bn2_g) * S_BN, r2(bn2_b), g3_W, r3_W,
                           r2(r3_b), dis_t_col)
    acc3 = scat(up3, tri, tci)
    h3, up4 = _post_pre(acc3, up3, res3, dis_t_col, r2(g3_b),
                        r2(bn3_g) * S_BN, r2(bn3_b), g4_W, dis_t_col)
    acc4 = scat(up4, tri, tci)
    hT = _kpool(acc4, up4, h3, dis_t_col, r2(g4_b),
                r2(bn4_g) * S_BN, r2(bn4_b))

    out = _khead(hT, a1_W, r2(a1_b), a2_W, r2(ln_g), r2(ln_b),
                 c1_W, r2(c1_b), jnp.pad(c2_W, ((0, 0), (0, 4))),
                 jnp.pad(r2(c2_b), ((0, 0), (0, 4))))
    return out[:, :100]


# speedup vs baseline: 8.0701x; 2.8744x over previous
import functools
import math

import jax
import jax.numpy as jnp
from jax import lax
from jax.experimental import pallas as pl
from jax.experimental.pallas import tpu as pltpu
from jax.experimental.pallas import tpu_sc as plsc

B, T, N, C = 16, 50, 55, 6
NT = T * N
NTP = NT + 2
MP = B * NTP
RT = 1024
S_BN = 1.0 / math.sqrt(1.0 + 1e-5)
NSUB = 16

_f32 = jnp.float32
_i32 = jnp.int32


def _row(i):
    return (i, 0)


def _rep(i):
    return (0, 0)


def _gelu(v):
    return 0.5 * v * (1.0 + jax.lax.erf(v * (1.0 / math.sqrt(2.0))))


def _ka_body(x_ref, Wp_ref, bp_ref, g1W_ref, r1W_ref, r1b_ref, dis_ref,
             up1_ref, res1_ref):
    h0 = jnp.dot(x_ref[...], Wp_ref[...],
                 preferred_element_type=_f32) + bp_ref[...]
    up1_ref[...] = jnp.dot(h0, g1W_ref[...],
                           preferred_element_type=_f32) * dis_ref[...]
    res1_ref[...] = jnp.dot(h0, r1W_ref[...],
                            preferred_element_type=_f32) + r1b_ref[...]


def _ka(x, Wp, bp, g1W, r1W, r1b, dis_s_col):
    return pl.pallas_call(
        _ka_body,
        grid=(MP // RT,),
        in_specs=[
            pl.BlockSpec((RT, 8), _row),
            pl.BlockSpec((8, 64), _rep),
            pl.BlockSpec((1, 64), _rep),
            pl.BlockSpec((64, 128), _rep),
            pl.BlockSpec((64, 128), _rep),
            pl.BlockSpec((1, 128), _rep),
            pl.BlockSpec((RT, 1), _row),
        ],
        out_specs=[pl.BlockSpec((RT, 128), _row),
                   pl.BlockSpec((RT, 128), _row)],
        out_shape=[jax.ShapeDtypeStruct((MP, 128), _f32),
                   jax.ShapeDtypeStruct((MP, 128), _f32)],
    )(x, Wp, bp, g1W, r1W, r1b, dis_s_col)


def _post_pre_body(acc_ref, up_ref, res_ref, disa_ref, gb_ref, bng_ref,
                   bnb_ref, W_ref, disb_ref, h_ref, upn_ref):
    v = disa_ref[...] * (acc_ref[...] + up_ref[...]) + gb_ref[...]
    h = _gelu(v * bng_ref[...] + bnb_ref[...] + res_ref[...])
    h_ref[...] = h
    upn_ref[...] = jnp.dot(h, W_ref[...],
                           preferred_element_type=_f32) * disb_ref[...]


def _post_pre(acc, up, res, dis_a, gb, bng_eff, bnb, W, dis_b):
    Fin = up.shape[1]
    Fout = W.shape[1]
    return pl.pallas_call(
        _post_pre_body,
        grid=(MP // RT,),
        in_specs=[
            pl.BlockSpec((RT, Fin), _row),
            pl.BlockSpec((RT, Fin), _row),
            pl.BlockSpec((RT, Fin), _row),
            pl.BlockSpec((RT, 1), _row),
            pl.BlockSpec((1, Fin), _rep),
            pl.BlockSpec((1, Fin), _rep),
            pl.BlockSpec((1, Fin), _rep),
            pl.BlockSpec((Fin, Fout), _rep),
            pl.BlockSpec((RT, 1), _row),
        ],
        out_specs=[pl.BlockSpec((RT, Fin), _row),
                   pl.BlockSpec((RT, Fout), _row)],
        out_shape=[jax.ShapeDtypeStruct((MP, Fin), _f32),
                   jax.ShapeDtypeStruct((MP, Fout), _f32)],
    )(acc, up, res, dis_a, gb, bng_eff, bnb, W, dis_b)


def _post_pre2_body(acc_ref, up_ref, res_ref, disa_ref, gb_ref, bng_ref,
                    bnb_ref, W_ref, rW_ref, rb_ref, disb_ref,
                    upn_ref, resn_ref):
    v = disa_ref[...] * (acc_ref[...] + up_ref[...]) + gb_ref[...]
    h = _gelu(v * bng_ref[...] + bnb_ref[...] + res_ref[...])
    upn_ref[...] = jnp.dot(h, W_ref[...],
                           preferred_element_type=_f32) * disb_ref[...]
    resn_ref[...] = jnp.dot(h, rW_ref[...],
                            preferred_element_type=_f32) + rb_ref[...]


def _post_pre2(acc, up, res, dis_a, gb, bng_eff, bnb, W, rW, rb, dis_b):
    Fin = up.shape[1]
    Fout = W.shape[1]
    return pl.pallas_call(
        _post_pre2_body,
        grid=(MP // RT,),
        in_specs=[
            pl.BlockSpec((RT, Fin), _row),
            pl.BlockSpec((RT, Fin), _row),
            pl.BlockSpec((RT, Fin), _row),
            pl.BlockSpec((RT, 1), _row),
            pl.BlockSpec((1, Fin), _rep),
            pl.BlockSpec((1, Fin), _rep),
            pl.BlockSpec((1, Fin), _rep),
            pl.BlockSpec((Fin, Fout), _rep),
            pl.BlockSpec((Fin, Fout), _rep),
            pl.BlockSpec((1, Fout), _rep),
            pl.BlockSpec((RT, 1), _row),
        ],
        out_specs=[pl.BlockSpec((RT, Fout), _row),
                   pl.BlockSpec((RT, Fout), _row)],
        out_shape=[jax.ShapeDtypeStruct((MP, Fout), _f32),
                   jax.ShapeDtypeStruct((MP, Fout), _f32)],
    )(acc, up, res, dis_a, gb, bng_eff, bnb, W, rW, rb, dis_b)


def _kpool_body(acc_ref, up_ref, res_ref, dis_ref, gb_ref, bng_ref, bnb_ref,
                out_ref):
    v = dis_ref[...] * (acc_ref[...] + up_ref[...]) + gb_ref[...]
    h4 = _gelu(v * bng_ref[...] + bnb_ref[...] + res_ref[...])
    ti = jax.lax.broadcasted_iota(_i32, (56, NTP), 0)
    ri = jax.lax.broadcasted_iota(_i32, (56, NTP), 1)
    sel = jnp.where((ri // N == ti) & (ri < NT), 1.0 / N, 0.0).astype(_f32)
    out_ref[0] = jnp.dot(sel, h4, preferred_element_type=_f32)


def _kpool(acc, up, res, dis_t_col, gb, bng_eff, bnb):
    return pl.pallas_call(
        _kpool_body,
        grid=(B,),
        in_specs=[
            pl.BlockSpec((NTP, 256), _row),
            pl.BlockSpec((NTP, 256), _row),
            pl.BlockSpec((NTP, 256), _row),
            pl.BlockSpec((NTP, 1), _row),
            pl.BlockSpec((1, 256), _rep),
            pl.BlockSpec((1, 256), _rep),
            pl.BlockSpec((1, 256), _rep),
        ],
        out_specs=pl.BlockSpec((1, 56, 256), lambda i: (i, 0, 0)),
        out_shape=jax.ShapeDtypeStruct((B, 56, 256), _f32),
    )(acc, up, res, dis_t_col, gb, bng_eff, bnb)


def _khead_body(hT_ref, a1_ref, a1b_ref, a2_ref, lng_ref, lnb_ref,
                c1_ref, c1b_ref, c2_ref, c2b_ref, out_ref):
    ti = jax.lax.broadcasted_iota(_i32, (56, 1), 0)
    tmask = ti < T
    for b in range(B):
        x = hT_ref[b]
        t = jnp.tanh(jnp.dot(x, a1_ref[...],
                             preferred_element_type=_f32) + a1b_ref[...])
        logits = jnp.dot(t, a2_ref[...], preferred_element_type=_f32)
        logits = jnp.where(tmask, logits, -1e30)
        e = jnp.exp(logits - jnp.max(logits, axis=0, keepdims=True))
        e = jnp.where(tmask, e, 0.0)
        w = e / jnp.sum(e, axis=0, keepdims=True)
        pooled = jnp.sum(x * w, axis=0, keepdims=True)
        mu = jnp.mean(pooled, axis=1, keepdims=True)
        var = jnp.mean((pooled - mu) ** 2, axis=1, keepdims=True)
        z = (pooled - mu) * jax.lax.rsqrt(var + 1e-5) * lng_ref[...] \
            + lnb_ref[...]
        z1 = _gelu(jnp.dot(z, c1_ref[...],
                           preferred_element_type=_f32) + c1b_ref[...])
        out_ref[pl.ds(b, 1), :] = jnp.dot(
            z1, c2_ref[...], preferred_element_type=_f32) + c2b_ref[...]


def _khead(hT, a1, a1b, a2, lng, lnb, c1, c1b, c2, c2b):
    return pl.pallas_call(
        _khead_body,
        grid=(1,),
        in_specs=[
            pl.BlockSpec((B, 56, 256), lambda i: (0, 0, 0)),
            pl.BlockSpec((256, 64), _rep),
            pl.BlockSpec((1, 64), _rep),
            pl.BlockSpec((64, 1), _rep),
            pl.BlockSpec((1, 256), _rep),
            pl.BlockSpec((1, 256), _rep),
            pl.BlockSpec((256, 256), _rep),
            pl.BlockSpec((1, 256), _rep),
            pl.BlockSpec((256, 104), _rep),
            pl.BlockSpec((1, 104), _rep),
        ],
        out_specs=pl.BlockSpec((B, 104), _rep),
        out_shape=jax.ShapeDtypeStruct((B, 104), _f32),
    )(hT, a1, a1b, a2, lng, lnb, c1, c1b, c2, c2b)


RT2 = 344


def _tmm_body(A_ref, up_ref, out_ref):
    out_ref[0] = jnp.dot(A_ref[...], up_ref[0],
                         preferred_element_type=_f32)


def _tmm(A, up):
    up3 = up.reshape(B, NTP, 256)
    out = pl.pallas_call(
        _tmm_body,
        grid=(B, NTP // RT2),
        in_specs=[
            pl.BlockSpec((RT2, NTP), lambda b, i: (i, 0)),
            pl.BlockSpec((1, NTP, 256), lambda b, i: (b, 0, 0)),
        ],
        out_specs=pl.BlockSpec((1, RT2, 256), lambda b, i: (b, i, 0)),
        out_shape=jax.ShapeDtypeStruct((B, NTP, 256), _f32),
    )(A, up3)
    return out.reshape(MP, 256)


def _flat_edges(s_ei, t_ei):
    goffs = (jnp.arange(T, dtype=_i32) * N)[:, None]
    sr0 = (s_ei[0][None, :] + goffs).reshape(-1)
    sc0 = (s_ei[1][None, :] + goffs).reshape(-1)
    offs = jnp.arange(B, dtype=_i32) * NTP
    sri = (sr0[None, :] + offs[:, None]).reshape(-1)
    sci = (sc0[None, :] + offs[:, None]).reshape(-1)
    tri = (t_ei[0][None, :] + offs[:, None]).reshape(-1)
    tci = (t_ei[1][None, :] + offs[:, None]).reshape(-1)
    return sri, sci, tri, tci


def kernel(x, s_ei, t_ei, Wp, bp, g1_W, g1_b, bn1_g, bn1_b, r1_W, r1_b,
           g2_W, g2_b, bn2_g, bn2_b, g3_W, g3_b, bn3_g, bn3_b, r3_W, r3_b,
           g4_W, g4_b, bn4_g, bn4_b, a1_W, a1_b, a2_W, a2_b, ln_g, ln_b,
           c1_W, c1_b, c2_W, c2_b):
    xf = jnp.pad(x.reshape(B, NT, C), ((0, 0), (0, 2), (0, 2)))
    xf = xf.reshape(MP, 8)

    deg_s = jnp.zeros((N,), _f32).at[s_ei[1]].add(1.0)
    deg_t = jnp.zeros((NT,), _f32).at[t_ei[1]].add(1.0)
    dis_s = jax.lax.rsqrt(deg_s + 1.0)
    dis_t = jax.lax.rsqrt(deg_t + 1.0)
    pad1 = jnp.ones((2,), _f32)
    dis_s_col = jnp.tile(jnp.concatenate([jnp.tile(dis_s, T), pad1]),
                         B)[:, None]
    dis_t_col = jnp.tile(jnp.concatenate([dis_t, pad1]), B)[:, None]

    sri, sci, tri, tci = _flat_edges(s_ei, t_ei)
    At = jnp.zeros((NTP, NTP), _f32).at[t_ei[1], t_ei[0]].add(1.0)

    def scat(up, r, c):
        return jnp.zeros(up.shape, _f32).at[c].add(up[r])

    def r2(v):
        return v[None, :]

    up1, res1 = _ka(xf, jnp.pad(Wp, ((0, 2), (0, 0))), r2(bp),
                    g1_W, r1_W, r2(r1_b), dis_s_col)
    acc1 = scat(up1, sri, sci)
    h1, up2 = _post_pre(acc1, up1, res1, dis_s_col, r2(g1_b),
                        r2(bn1_g) * S_BN, r2(bn1_b), g2_W, dis_s_col)
    acc2 = scat(up2, sri, sci)
    up3, res3 = _post_pre2(acc2, up2, h1, dis_s_col, r2(g2_b),
                           r2(bn2_g) * S_BN, r2(bn2_b), g3_W, r3_W,
                           r2(r3_b), dis_t_col)
    acc3 = _tmm(At, up3)
    h3, up4 = _post_pre(acc3, up3, res3, dis_t_col, r2(g3_b),
                        r2(bn3_g) * S_BN, r2(bn3_b), g4_W, dis_t_col)
    acc4 = _tmm(At, up4)
    hT = _kpool(acc4, up4, h3, dis_t_col, r2(g4_b),
                r2(bn4_g) * S_BN, r2(bn4_b))

    out = _khead(hT, a1_W, r2(a1_b), a2_W, r2(ln_g), r2(ln_b),
                 c1_W, r2(c1_b), jnp.pad(c2_W, ((0, 0), (0, 4))),
                 jnp.pad(r2(c2_b), ((0, 0), (0, 4))))
    return out[:, :100]


# speedup vs baseline: 8.1961x; 1.0156x over previous
import functools
import math

import jax
import jax.numpy as jnp
from jax import lax
from jax.experimental import pallas as pl
from jax.experimental.pallas import tpu as pltpu
from jax.experimental.pallas import tpu_sc as plsc

B, T, N, C = 16, 50, 55, 6
NT = T * N
NTP = NT + 2
MP = B * NTP
RT = 1024
S_BN = 1.0 / math.sqrt(1.0 + 1e-5)
NSUB = 16

_f32 = jnp.float32
_i32 = jnp.int32


def _row(i):
    return (i, 0)


def _rep(i):
    return (0, 0)


def _gelu(v):
    return 0.5 * v * (1.0 + jax.lax.erf(v * (1.0 / math.sqrt(2.0))))


def _ka_body(x_ref, Wp_ref, bp_ref, g1W_ref, r1W_ref, r1b_ref, dis_ref,
             up1_ref, res1_ref):
    h0 = jnp.dot(x_ref[...], Wp_ref[...],
                 preferred_element_type=_f32) + bp_ref[...]
    up1_ref[...] = jnp.dot(h0, g1W_ref[...],
                           preferred_element_type=_f32) * dis_ref[...]
    res1_ref[...] = jnp.dot(h0, r1W_ref[...],
                            preferred_element_type=_f32) + r1b_ref[...]


def _ka(x, Wp, bp, g1W, r1W, r1b, dis_s_col):
    return pl.pallas_call(
        _ka_body,
        grid=(MP // RT,),
        in_specs=[
            pl.BlockSpec((RT, 8), _row),
            pl.BlockSpec((8, 64), _rep),
            pl.BlockSpec((1, 64), _rep),
            pl.BlockSpec((64, 128), _rep),
            pl.BlockSpec((64, 128), _rep),
            pl.BlockSpec((1, 128), _rep),
            pl.BlockSpec((RT, 1), _row),
        ],
        out_specs=[pl.BlockSpec((RT, 128), _row),
                   pl.BlockSpec((RT, 128), _row)],
        out_shape=[jax.ShapeDtypeStruct((MP, 128), _f32),
                   jax.ShapeDtypeStruct((MP, 128), _f32)],
    )(x, Wp, bp, g1W, r1W, r1b, dis_s_col)


def _post_pre_body(acc_ref, up_ref, res_ref, disa_ref, gb_ref, bng_ref,
                   bnb_ref, W_ref, disb_ref, h_ref, upn_ref):
    v = disa_ref[...] * (acc_ref[...] + up_ref[...]) + gb_ref[...]
    h = _gelu(v * bng_ref[...] + bnb_ref[...] + res_ref[...])
    h_ref[...] = h
    upn_ref[...] = jnp.dot(h, W_ref[...],
                           preferred_element_type=_f32) * disb_ref[...]


def _post_pre(acc, up, res, dis_a, gb, bng_eff, bnb, W, dis_b):
    Fin = up.shape[1]
    Fout = W.shape[1]
    return pl.pallas_call(
        _post_pre_body,
        grid=(MP // RT,),
        in_specs=[
            pl.BlockSpec((RT, Fin), _row),
            pl.BlockSpec((RT, Fin), _row),
            pl.BlockSpec((RT, Fin), _row),
            pl.BlockSpec((RT, 1), _row),
            pl.BlockSpec((1, Fin), _rep),
            pl.BlockSpec((1, Fin), _rep),
            pl.BlockSpec((1, Fin), _rep),
            pl.BlockSpec((Fin, Fout), _rep),
            pl.BlockSpec((RT, 1), _row),
        ],
        out_specs=[pl.BlockSpec((RT, Fin), _row),
                   pl.BlockSpec((RT, Fout), _row)],
        out_shape=[jax.ShapeDtypeStruct((MP, Fin), _f32),
                   jax.ShapeDtypeStruct((MP, Fout), _f32)],
    )(acc, up, res, dis_a, gb, bng_eff, bnb, W, dis_b)


def _post_pre2_body(acc_ref, up_ref, res_ref, disa_ref, gb_ref, bng_ref,
                    bnb_ref, W_ref, rW_ref, rb_ref, disb_ref,
                    upn_ref, resn_ref):
    v = disa_ref[...] * (acc_ref[...] + up_ref[...]) + gb_ref[...]
    h = _gelu(v * bng_ref[...] + bnb_ref[...] + res_ref[...])
    upn_ref[...] = jnp.dot(h, W_ref[...],
                           preferred_element_type=_f32) * disb_ref[...]
    resn_ref[...] = jnp.dot(h, rW_ref[...],
                            preferred_element_type=_f32) + rb_ref[...]


def _post_pre2(acc, up, res, dis_a, gb, bng_eff, bnb, W, rW, rb, dis_b):
    Fin = up.shape[1]
    Fout = W.shape[1]
    return pl.pallas_call(
        _post_pre2_body,
        grid=(MP // RT,),
        in_specs=[
            pl.BlockSpec((RT, Fin), _row),
            pl.BlockSpec((RT, Fin), _row),
            pl.BlockSpec((RT, Fin), _row),
            pl.BlockSpec((RT, 1), _row),
            pl.BlockSpec((1, Fin), _rep),
            pl.BlockSpec((1, Fin), _rep),
            pl.BlockSpec((1, Fin), _rep),
            pl.BlockSpec((Fin, Fout), _rep),
            pl.BlockSpec((Fin, Fout), _rep),
            pl.BlockSpec((1, Fout), _rep),
            pl.BlockSpec((RT, 1), _row),
        ],
        out_specs=[pl.BlockSpec((RT, Fout), _row),
                   pl.BlockSpec((RT, Fout), _row)],
        out_shape=[jax.ShapeDtypeStruct((MP, Fout), _f32),
                   jax.ShapeDtypeStruct((MP, Fout), _f32)],
    )(acc, up, res, dis_a, gb, bng_eff, bnb, W, rW, rb, dis_b)


def _kpool_body(acc_ref, up_ref, res_ref, dis_ref, gb_ref, bng_ref, bnb_ref,
                out_ref):
    v = dis_ref[...] * (acc_ref[...] + up_ref[...]) + gb_ref[...]
    h4 = _gelu(v * bng_ref[...] + bnb_ref[...] + res_ref[...])
    ti = jax.lax.broadcasted_iota(_i32, (56, NTP), 0)
    ri = jax.lax.broadcasted_iota(_i32, (56, NTP), 1)
    sel = jnp.where((ri // N == ti) & (ri < NT), 1.0 / N, 0.0).astype(_f32)
    out_ref[0] = jnp.dot(sel, h4, preferred_element_type=_f32)


def _kpool(acc, up, res, dis_t_col, gb, bng_eff, bnb):
    return pl.pallas_call(
        _kpool_body,
        grid=(B,),
        in_specs=[
            pl.BlockSpec((NTP, 256), _row),
            pl.BlockSpec((NTP, 256), _row),
            pl.BlockSpec((NTP, 256), _row),
            pl.BlockSpec((NTP, 1), _row),
            pl.BlockSpec((1, 256), _rep),
            pl.BlockSpec((1, 256), _rep),
            pl.BlockSpec((1, 256), _rep),
        ],
        out_specs=pl.BlockSpec((1, 56, 256), lambda i: (i, 0, 0)),
        out_shape=jax.ShapeDtypeStruct((B, 56, 256), _f32),
    )(acc, up, res, dis_t_col, gb, bng_eff, bnb)


def _khead_body(hT_ref, a1_ref, a1b_ref, a2_ref, lng_ref, lnb_ref,
                c1_ref, c1b_ref, c2_ref, c2b_ref, out_ref):
    ti = jax.lax.broadcasted_iota(_i32, (56, 1), 0)
    tmask = ti < T
    for b in range(B):
        x = hT_ref[b]
        t = jnp.tanh(jnp.dot(x, a1_ref[...],
                             preferred_element_type=_f32) + a1b_ref[...])
        logits = jnp.dot(t, a2_ref[...], preferred_element_type=_f32)
        logits = jnp.where(tmask, logits, -1e30)
        e = jnp.exp(logits - jnp.max(logits, axis=0, keepdims=True))
        e = jnp.where(tmask, e, 0.0)
        w = e / jnp.sum(e, axis=0, keepdims=True)
        pooled = jnp.sum(x * w, axis=0, keepdims=True)
        mu = jnp.mean(pooled, axis=1, keepdims=True)
        var = jnp.mean((pooled - mu) ** 2, axis=1, keepdims=True)
        z = (pooled - mu) * jax.lax.rsqrt(var + 1e-5) * lng_ref[...] \
            + lnb_ref[...]
        z1 = _gelu(jnp.dot(z, c1_ref[...],
                           preferred_element_type=_f32) + c1b_ref[...])
        out_ref[pl.ds(b, 1), :] = jnp.dot(
            z1, c2_ref[...], preferred_element_type=_f32) + c2b_ref[...]


def _khead(hT, a1, a1b, a2, lng, lnb, c1, c1b, c2, c2b):
    return pl.pallas_call(
        _khead_body,
        grid=(1,),
        in_specs=[
            pl.BlockSpec((B, 56, 256), lambda i: (0, 0, 0)),
            pl.BlockSpec((256, 64), _rep),
            pl.BlockSpec((1, 64), _rep),
            pl.BlockSpec((64, 1), _rep),
            pl.BlockSpec((1, 256), _rep),
            pl.BlockSpec((1, 256), _rep),
            pl.BlockSpec((256, 256), _rep),
            pl.BlockSpec((1, 256), _rep),
            pl.BlockSpec((256, 104), _rep),
            pl.BlockSpec((1, 104), _rep),
        ],
        out_specs=pl.BlockSpec((B, 104), _rep),
        out_shape=jax.ShapeDtypeStruct((B, 104), _f32),
    )(hT, a1, a1b, a2, lng, lnb, c1, c1b, c2, c2b)


RT2 = 344


def _tmm_body(A_ref, up_ref, out_ref):
    out_ref[0] = jnp.dot(A_ref[...], up_ref[0],
                         preferred_element_type=_f32)


def _tmm(A, up):
    F = up.shape[1]
    up3 = up.reshape(B, NTP, F)
    out = pl.pallas_call(
        _tmm_body,
        grid=(B, NTP // RT2),
        in_specs=[
            pl.BlockSpec((RT2, NTP), lambda b, i: (i, 0)),
            pl.BlockSpec((1, NTP, F), lambda b, i: (b, 0, 0)),
        ],
        out_specs=pl.BlockSpec((1, RT2, F), lambda b, i: (b, i, 0)),
        out_shape=jax.ShapeDtypeStruct((B, NTP, F), _f32),
    )(A, up3)
    return out.reshape(MP, F)


def _flat_edges(s_ei, t_ei):
    goffs = (jnp.arange(T, dtype=_i32) * N)[:, None]
    sr0 = (s_ei[0][None, :] + goffs).reshape(-1)
    sc0 = (s_ei[1][None, :] + goffs).reshape(-1)
    offs = jnp.arange(B, dtype=_i32) * NTP
    sri = (sr0[None, :] + offs[:, None]).reshape(-1)
    sci = (sc0[None, :] + offs[:, None]).reshape(-1)
    tri = (t_ei[0][None, :] + offs[:, None]).reshape(-1)
    tci = (t_ei[1][None, :] + offs[:, None]).reshape(-1)
    return sri, sci, tri, tci


def kernel(x, s_ei, t_ei, Wp, bp, g1_W, g1_b, bn1_g, bn1_b, r1_W, r1_b,
           g2_W, g2_b, bn2_g, bn2_b, g3_W, g3_b, bn3_g, bn3_b, r3_W, r3_b,
           g4_W, g4_b, bn4_g, bn4_b, a1_W, a1_b, a2_W, a2_b, ln_g, ln_b,
           c1_W, c1_b, c2_W, c2_b):
    xf = jnp.pad(x.reshape(B, NT, C), ((0, 0), (0, 2), (0, 2)))
    xf = xf.reshape(MP, 8)

    deg_s = jnp.zeros((N,), _f32).at[s_ei[1]].add(1.0)
    deg_t = jnp.zeros((NT,), _f32).at[t_ei[1]].add(1.0)
    dis_s = jax.lax.rsqrt(deg_s + 1.0)
    dis_t = jax.lax.rsqrt(deg_t + 1.0)
    pad1 = jnp.ones((2,), _f32)
    dis_s_col = jnp.tile(jnp.concatenate([jnp.tile(dis_s, T), pad1]),
                         B)[:, None]
    dis_t_col = jnp.tile(jnp.concatenate([dis_t, pad1]), B)[:, None]

    At = jnp.zeros((NTP, NTP), _f32).at[t_ei[1], t_ei[0]].add(1.0)
    goffs = (jnp.arange(T, dtype=_i32) * N)[:, None]
    sr0 = (s_ei[0][None, :] + goffs).reshape(-1)
    sc0 = (s_ei[1][None, :] + goffs).reshape(-1)
    As = jnp.zeros((NTP, NTP), _f32).at[sc0, sr0].add(1.0)

    def r2(v):
        return v[None, :]

    up1, res1 = _ka(xf, jnp.pad(Wp, ((0, 2), (0, 0))), r2(bp),
                    g1_W, r1_W, r2(r1_b), dis_s_col)
    acc1 = _tmm(As, up1)
    h1, up2 = _post_pre(acc1, up1, res1, dis_s_col, r2(g1_b),
                        r2(bn1_g) * S_BN, r2(bn1_b), g2_W, dis_s_col)
    acc2 = _tmm(As, up2)
    up3, res3 = _post_pre2(acc2, up2, h1, dis_s_col, r2(g2_b),
                           r2(bn2_g) * S_BN, r2(bn2_b), g3_W, r3_W,
                           r2(r3_b), dis_t_col)
    acc3 = _tmm(At, up3)
    h3, up4 = _post_pre(acc3, up3, res3, dis_t_col, r2(g3_b),
                        r2(bn3_g) * S_BN, r2(bn3_b), g4_W, dis_t_col)
    acc4 = _tmm(At, up4)
    hT = _kpool(acc4, up4, h3, dis_t_col, r2(g4_b),
                r2(bn4_g) * S_BN, r2(bn4_b))

    out = _khead(hT, a1_W, r2(a1_b), a2_W, r2(ln_g), r2(ln_b),
                 c1_W, r2(c1_b), jnp.pad(c2_W, ((0, 0), (0, 4))),
                 jnp.pad(r2(c2_b), ((0, 0), (0, 4))))
    return out[:, :100]


# speedup vs baseline: 8.2036x; 1.0009x over previous
import math

import jax
import jax.numpy as jnp
from jax.experimental import pallas as pl

B, T, N, C = 16, 50, 55, 6
NT = T * N
NTP = NT + 2
MP = B * NTP
RT = 1024
S_BN = 1.0 / math.sqrt(1.0 + 1e-5)

_f32 = jnp.float32
_i32 = jnp.int32


def _row(i):
    return (i, 0)


def _rep(i):
    return (0, 0)


def _gelu(v):
    return 0.5 * v * (1.0 + jax.lax.erf(v * (1.0 / math.sqrt(2.0))))


def _ka_body(x_ref, Wp_ref, bp_ref, g1W_ref, r1W_ref, r1b_ref, dis_ref,
             up1_ref, res1_ref):
    h0 = jnp.dot(x_ref[...], Wp_ref[...],
                 preferred_element_type=_f32) + bp_ref[...]
    up1_ref[...] = jnp.dot(h0, g1W_ref[...],
                           preferred_element_type=_f32) * dis_ref[...]
    res1_ref[...] = jnp.dot(h0, r1W_ref[...],
                            preferred_element_type=_f32) + r1b_ref[...]


def _ka(x, Wp, bp, g1W, r1W, r1b, dis_s_col):
    return pl.pallas_call(
        _ka_body,
        grid=(MP // RT,),
        in_specs=[
            pl.BlockSpec((RT, 8), _row),
            pl.BlockSpec((8, 64), _rep),
            pl.BlockSpec((1, 64), _rep),
            pl.BlockSpec((64, 128), _rep),
            pl.BlockSpec((64, 128), _rep),
            pl.BlockSpec((1, 128), _rep),
            pl.BlockSpec((RT, 1), _row),
        ],
        out_specs=[pl.BlockSpec((RT, 128), _row),
                   pl.BlockSpec((RT, 128), _row)],
        out_shape=[jax.ShapeDtypeStruct((MP, 128), _f32),
                   jax.ShapeDtypeStruct((MP, 128), _f32)],
    )(x, Wp, bp, g1W, r1W, r1b, dis_s_col)


def _post_pre_body(acc_ref, up_ref, res_ref, disa_ref, gb_ref, bng_ref,
                   bnb_ref, W_ref, disb_ref, h_ref, upn_ref):
    v = disa_ref[...] * (acc_ref[...] + up_ref[...]) + gb_ref[...]
    h = _gelu(v * bng_ref[...] + bnb_ref[...] + res_ref[...])
    h_ref[...] = h
    upn_ref[...] = jnp.dot(h, W_ref[...],
                           preferred_element_type=_f32) * disb_ref[...]


def _post_pre(acc, up, res, dis_a, gb, bng_eff, bnb, W, dis_b):
    Fin = up.shape[1]
    Fout = W.shape[1]
    return pl.pallas_call(
        _post_pre_body,
        grid=(MP // RT,),
        in_specs=[
            pl.BlockSpec((RT, Fin), _row),
            pl.BlockSpec((RT, Fin), _row),
            pl.BlockSpec((RT, Fin), _row),
            pl.BlockSpec((RT, 1), _row),
            pl.BlockSpec((1, Fin), _rep),
            pl.BlockSpec((1, Fin), _rep),
            pl.BlockSpec((1, Fin), _rep),
            pl.BlockSpec((Fin, Fout), _rep),
            pl.BlockSpec((RT, 1), _row),
        ],
        out_specs=[pl.BlockSpec((RT, Fin), _row),
                   pl.BlockSpec((RT, Fout), _row)],
        out_shape=[jax.ShapeDtypeStruct((MP, Fin), _f32),
                   jax.ShapeDtypeStruct((MP, Fout), _f32)],
    )(acc, up, res, dis_a, gb, bng_eff, bnb, W, dis_b)


def _post_pre2_body(acc_ref, up_ref, res_ref, disa_ref, gb_ref, bng_ref,
                    bnb_ref, W_ref, rW_ref, rb_ref, disb_ref,
                    upn_ref, resn_ref):
    v = disa_ref[...] * (acc_ref[...] + up_ref[...]) + gb_ref[...]
    h = _gelu(v * bng_ref[...] + bnb_ref[...] + res_ref[...])
    upn_ref[...] = jnp.dot(h, W_ref[...],
                           preferred_element_type=_f32) * disb_ref[...]
    resn_ref[...] = jnp.dot(h, rW_ref[...],
                            preferred_element_type=_f32) + rb_ref[...]


def _post_pre2(acc, up, res, dis_a, gb, bng_eff, bnb, W, rW, rb, dis_b):
    Fin = up.shape[1]
    Fout = W.shape[1]
    return pl.pallas_call(
        _post_pre2_body,
        grid=(MP // RT,),
        in_specs=[
            pl.BlockSpec((RT, Fin), _row),
            pl.BlockSpec((RT, Fin), _row),
            pl.BlockSpec((RT, Fin), _row),
            pl.BlockSpec((RT, 1), _row),
            pl.BlockSpec((1, Fin), _rep),
            pl.BlockSpec((1, Fin), _rep),
            pl.BlockSpec((1, Fin), _rep),
            pl.BlockSpec((Fin, Fout), _rep),
            pl.BlockSpec((Fin, Fout), _rep),
            pl.BlockSpec((1, Fout), _rep),
            pl.BlockSpec((RT, 1), _row),
        ],
        out_specs=[pl.BlockSpec((RT, Fout), _row),
                   pl.BlockSpec((RT, Fout), _row)],
        out_shape=[jax.ShapeDtypeStruct((MP, Fout), _f32),
                   jax.ShapeDtypeStruct((MP, Fout), _f32)],
    )(acc, up, res, dis_a, gb, bng_eff, bnb, W, rW, rb, dis_b)


def _kpool_body(acc_ref, up_ref, res_ref, dis_ref, gb_ref, bng_ref, bnb_ref,
                out_ref):
    v = dis_ref[...] * (acc_ref[...] + up_ref[...]) + gb_ref[...]
    h4 = _gelu(v * bng_ref[...] + bnb_ref[...] + res_ref[...])
    ti = jax.lax.broadcasted_iota(_i32, (56, NTP), 0)
    ri = jax.lax.broadcasted_iota(_i32, (56, NTP), 1)
    sel = jnp.where((ri // N == ti) & (ri < NT), 1.0 / N, 0.0).astype(_f32)
    out_ref[0] = jnp.dot(sel, h4, preferred_element_type=_f32)


def _kpool(acc, up, res, dis_t_col, gb, bng_eff, bnb):
    return pl.pallas_call(
        _kpool_body,
        grid=(B,),
        in_specs=[
            pl.BlockSpec((NTP, 256), _row),
            pl.BlockSpec((NTP, 256), _row),
            pl.BlockSpec((NTP, 256), _row),
            pl.BlockSpec((NTP, 1), _row),
            pl.BlockSpec((1, 256), _rep),
            pl.BlockSpec((1, 256), _rep),
            pl.BlockSpec((1, 256), _rep),
        ],
        out_specs=pl.BlockSpec((1, 56, 256), lambda i: (i, 0, 0)),
        out_shape=jax.ShapeDtypeStruct((B, 56, 256), _f32),
    )(acc, up, res, dis_t_col, gb, bng_eff, bnb)


def _khead_body(hT_ref, a1_ref, a1b_ref, a2_ref, lng_ref, lnb_ref,
                c1_ref, c1b_ref, c2_ref, c2b_ref, out_ref):
    ti = jax.lax.broadcasted_iota(_i32, (56, 1), 0)
    tmask = ti < T
    for b in range(B):
        x = hT_ref[b]
        t = jnp.tanh(jnp.dot(x, a1_ref[...],
                             preferred_element_type=_f32) + a1b_ref[...])
        logits = jnp.dot(t, a2_ref[...], preferred_element_type=_f32)
        logits = jnp.where(tmask, logits, -1e30)
        e = jnp.exp(logits - jnp.max(logits, axis=0, keepdims=True))
        e = jnp.where(tmask, e, 0.0)
        w = e / jnp.sum(e, axis=0, keepdims=True)
        pooled = jnp.sum(x * w, axis=0, keepdims=True)
        mu = jnp.mean(pooled, axis=1, keepdims=True)
        var = jnp.mean((pooled - mu) ** 2, axis=1, keepdims=True)
        z = (pooled - mu) * jax.lax.rsqrt(var + 1e-5) * lng_ref[...] \
            + lnb_ref[...]
        z1 = _gelu(jnp.dot(z, c1_ref[...],
                           preferred_element_type=_f32) + c1b_ref[...])
        out_ref[pl.ds(b, 1), :] = jnp.dot(
            z1, c2_ref[...], preferred_element_type=_f32) + c2b_ref[...]


def _khead(hT, a1, a1b, a2, lng, lnb, c1, c1b, c2, c2b):
    return pl.pallas_call(
        _khead_body,
        grid=(1,),
        in_specs=[
            pl.BlockSpec((B, 56, 256), lambda i: (0, 0, 0)),
            pl.BlockSpec((256, 64), _rep),
            pl.BlockSpec((1, 64), _rep),
            pl.BlockSpec((64, 1), _rep),
            pl.BlockSpec((1, 256), _rep),
            pl.BlockSpec((1, 256), _rep),
            pl.BlockSpec((256, 256), _rep),
            pl.BlockSpec((1, 256), _rep),
            pl.BlockSpec((256, 104), _rep),
            pl.BlockSpec((1, 104), _rep),
        ],
        out_specs=pl.BlockSpec((B, 104), _rep),
        out_shape=jax.ShapeDtypeStruct((B, 104), _f32),
    )(hT, a1, a1b, a2, lng, lnb, c1, c1b, c2, c2b)


RT2 = 344


def _tmm_body(A_ref, up_ref, out_ref):
    out_ref[0] = jnp.dot(A_ref[...], up_ref[0],
                         preferred_element_type=_f32)


def _tmm(A, up):
    F = up.shape[1]
    up3 = up.reshape(B, NTP, F)
    out = pl.pallas_call(
        _tmm_body,
        grid=(B, NTP // RT2),
        in_specs=[
            pl.BlockSpec((RT2, NTP), lambda b, i: (i, 0)),
            pl.BlockSpec((1, NTP, F), lambda b, i: (b, 0, 0)),
        ],
        out_specs=pl.BlockSpec((1, RT2, F), lambda b, i: (b, i, 0)),
        out_shape=jax.ShapeDtypeStruct((B, NTP, F), _f32),
    )(A, up3)
    return out.reshape(MP, F)


def kernel(x, s_ei, t_ei, Wp, bp, g1_W, g1_b, bn1_g, bn1_b, r1_W, r1_b,
           g2_W, g2_b, bn2_g, bn2_b, g3_W, g3_b, bn3_g, bn3_b, r3_W, r3_b,
           g4_W, g4_b, bn4_g, bn4_b, a1_W, a1_b, a2_W, a2_b, ln_g, ln_b,
           c1_W, c1_b, c2_W, c2_b):
    xf = jnp.pad(x.reshape(B, NT, C), ((0, 0), (0, 2), (0, 2)))
    xf = xf.reshape(MP, 8)

    deg_s = jnp.zeros((N,), _f32).at[s_ei[1]].add(1.0)
    deg_t = jnp.zeros((NT,), _f32).at[t_ei[1]].add(1.0)
    dis_s = jax.lax.rsqrt(deg_s + 1.0)
    dis_t = jax.lax.rsqrt(deg_t + 1.0)
    pad1 = jnp.ones((2,), _f32)
    dis_s_col = jnp.tile(jnp.concatenate([jnp.tile(dis_s, T), pad1]),
                         B)[:, None]
    dis_t_col = jnp.tile(jnp.concatenate([dis_t, pad1]), B)[:, None]

    At = jnp.zeros((NTP, NTP), _f32).at[t_ei[1], t_ei[0]].add(1.0)
    goffs = (jnp.arange(T, dtype=_i32) * N)[:, None]
    sr0 = (s_ei[0][None, :] + goffs).reshape(-1)
    sc0 = (s_ei[1][None, :] + goffs).reshape(-1)
    As = jnp.zeros((NTP, NTP), _f32).at[sc0, sr0].add(1.0)

    def r2(v):
        return v[None, :]

    up1, res1 = _ka(xf, jnp.pad(Wp, ((0, 2), (0, 0))), r2(bp),
                    g1_W, r1_W, r2(r1_b), dis_s_col)
    acc1 = _tmm(As, up1)
    h1, up2 = _post_pre(acc1, up1, res1, dis_s_col, r2(g1_b),
                        r2(bn1_g) * S_BN, r2(bn1_b), g2_W, dis_s_col)
    acc2 = _tmm(As, up2)
    up3, res3 = _post_pre2(acc2, up2, h1, dis_s_col, r2(g2_b),
                           r2(bn2_g) * S_BN, r2(bn2_b), g3_W, r3_W,
                           r2(r3_b), dis_t_col)
    acc3 = _tmm(At, up3)
    h3, up4 = _post_pre(acc3, up3, res3, dis_t_col, r2(g3_b),
                        r2(bn3_g) * S_BN, r2(bn3_b), g4_W, dis_t_col)
    acc4 = _tmm(At, up4)
    hT = _kpool(acc4, up4, h3, dis_t_col, r2(g4_b),
                r2(bn4_g) * S_BN, r2(bn4_b))

    out = _khead(hT, a1_W, r2(a1_b), a2_W, r2(ln_g), r2(ln_b),
                 c1_W, r2(c1_b), jnp.pad(c2_W, ((0, 0), (0, 4))),
                 jnp.pad(r2(c2_b), ((0, 0), (0, 4))))
    return out[:, :100]


# speedup vs baseline: 10.8679x; 1.3248x over previous
import math

import jax
import jax.numpy as jnp
from jax.experimental import pallas as pl

B, T, N, C = 16, 50, 55, 6
NT = T * N
NTP = NT + 2
MP = B * NTP
RT = 1024
S_BN = 1.0 / math.sqrt(1.0 + 1e-5)

_f32 = jnp.float32
_i32 = jnp.int32


def _row(i):
    return (i, 0)


def _rep(i):
    return (0, 0)


def _gelu(v):
    return 0.5 * v * (1.0 + jax.lax.erf(v * (1.0 / math.sqrt(2.0))))


def _ka_body(x_ref, Wp_ref, bp_ref, g1W_ref, r1W_ref, r1b_ref, dis_ref,
             up1_ref, res1_ref):
    h0 = jnp.dot(x_ref[...], Wp_ref[...],
                 preferred_element_type=_f32) + bp_ref[...]
    up1_ref[...] = jnp.dot(h0, g1W_ref[...],
                           preferred_element_type=_f32) * dis_ref[...]
    res1_ref[...] = jnp.dot(h0, r1W_ref[...],
                            preferred_element_type=_f32) + r1b_ref[...]


def _ka(x, Wp, bp, g1W, r1W, r1b, dis_s_col):
    return pl.pallas_call(
        _ka_body,
        grid=(MP // RT,),
        in_specs=[
            pl.BlockSpec((RT, 8), _row),
            pl.BlockSpec((8, 64), _rep),
            pl.BlockSpec((1, 64), _rep),
            pl.BlockSpec((64, 128), _rep),
            pl.BlockSpec((64, 128), _rep),
            pl.BlockSpec((1, 128), _rep),
            pl.BlockSpec((RT, 1), _row),
        ],
        out_specs=[pl.BlockSpec((RT, 128), _row),
                   pl.BlockSpec((RT, 128), _row)],
        out_shape=[jax.ShapeDtypeStruct((MP, 128), _f32),
                   jax.ShapeDtypeStruct((MP, 128), _f32)],
    )(x, Wp, bp, g1W, r1W, r1b, dis_s_col)


def _post_pre_body(acc_ref, up_ref, res_ref, disa_ref, gb_ref, bng_ref,
                   bnb_ref, W_ref, disb_ref, h_ref, upn_ref):
    v = disa_ref[...] * (acc_ref[...] + up_ref[...]) + gb_ref[...]
    h = _gelu(v * bng_ref[...] + bnb_ref[...] + res_ref[...])
    h_ref[...] = h
    upn_ref[...] = jnp.dot(h, W_ref[...],
                           preferred_element_type=_f32) * disb_ref[...]


def _post_pre(acc, up, res, dis_a, gb, bng_eff, bnb, W, dis_b):
    Fin = up.shape[1]
    Fout = W.shape[1]
    return pl.pallas_call(
        _post_pre_body,
        grid=(MP // RT,),
        in_specs=[
            pl.BlockSpec((RT, Fin), _row),
            pl.BlockSpec((RT, Fin), _row),
            pl.BlockSpec((RT, Fin), _row),
            pl.BlockSpec((RT, 1), _row),
            pl.BlockSpec((1, Fin), _rep),
            pl.BlockSpec((1, Fin), _rep),
            pl.BlockSpec((1, Fin), _rep),
            pl.BlockSpec((Fin, Fout), _rep),
            pl.BlockSpec((RT, 1), _row),
        ],
        out_specs=[pl.BlockSpec((RT, Fin), _row),
                   pl.BlockSpec((RT, Fout), _row)],
        out_shape=[jax.ShapeDtypeStruct((MP, Fin), _f32),
                   jax.ShapeDtypeStruct((MP, Fout), _f32)],
    )(acc, up, res, dis_a, gb, bng_eff, bnb, W, dis_b)


def _post_pre2_body(acc_ref, up_ref, res_ref, disa_ref, gb_ref, bng_ref,
                    bnb_ref, W_ref, rW_ref, rb_ref, disb_ref,
                    upn_ref, resn_ref):
    v = disa_ref[...] * (acc_ref[...] + up_ref[...]) + gb_ref[...]
    h = _gelu(v * bng_ref[...] + bnb_ref[...] + res_ref[...])
    upn_ref[...] = jnp.dot(h, W_ref[...],
                           preferred_element_type=_f32) * disb_ref[...]
    resn_ref[...] = jnp.dot(h, rW_ref[...],
                            preferred_element_type=_f32) + rb_ref[...]


def _post_pre2(acc, up, res, dis_a, gb, bng_eff, bnb, W, rW, rb, dis_b):
    Fin = up.shape[1]
    Fout = W.shape[1]
    return pl.pallas_call(
        _post_pre2_body,
        grid=(MP // RT,),
        in_specs=[
            pl.BlockSpec((RT, Fin), _row),
            pl.BlockSpec((RT, Fin), _row),
            pl.BlockSpec((RT, Fin), _row),
            pl.BlockSpec((RT, 1), _row),
            pl.BlockSpec((1, Fin), _rep),
            pl.BlockSpec((1, Fin), _rep),
            pl.BlockSpec((1, Fin), _rep),
            pl.BlockSpec((Fin, Fout), _rep),
            pl.BlockSpec((Fin, Fout), _rep),
            pl.BlockSpec((1, Fout), _rep),
            pl.BlockSpec((RT, 1), _row),
        ],
        out_specs=[pl.BlockSpec((RT, Fout), _row),
                   pl.BlockSpec((RT, Fout), _row)],
        out_shape=[jax.ShapeDtypeStruct((MP, Fout), _f32),
                   jax.ShapeDtypeStruct((MP, Fout), _f32)],
    )(acc, up, res, dis_a, gb, bng_eff, bnb, W, rW, rb, dis_b)


def _kpool_body(acc_ref, up_ref, res_ref, dis_ref, gb_ref, bng_ref, bnb_ref,
                out_ref):
    v = dis_ref[...] * (acc_ref[...] + up_ref[...]) + gb_ref[...]
    h4 = _gelu(v * bng_ref[...] + bnb_ref[...] + res_ref[...])
    ti = jax.lax.broadcasted_iota(_i32, (56, NTP), 0)
    ri = jax.lax.broadcasted_iota(_i32, (56, NTP), 1)
    sel = jnp.where((ri // N == ti) & (ri < NT), 1.0 / N, 0.0).astype(_f32)
    out_ref[0] = jnp.dot(sel, h4, preferred_element_type=_f32)


def _kpool(acc, up, res, dis_t_col, gb, bng_eff, bnb):
    return pl.pallas_call(
        _kpool_body,
        grid=(B,),
        in_specs=[
            pl.BlockSpec((NTP, 256), _row),
            pl.BlockSpec((NTP, 256), _row),
            pl.BlockSpec((NTP, 256), _row),
            pl.BlockSpec((NTP, 1), _row),
            pl.BlockSpec((1, 256), _rep),
            pl.BlockSpec((1, 256), _rep),
            pl.BlockSpec((1, 256), _rep),
        ],
        out_specs=pl.BlockSpec((1, 56, 256), lambda i: (i, 0, 0)),
        out_shape=jax.ShapeDtypeStruct((B, 56, 256), _f32),
    )(acc, up, res, dis_t_col, gb, bng_eff, bnb)


def _khead_body(hT_ref, a1_ref, a1b_ref, a2_ref, lng_ref, lnb_ref,
                c1_ref, c1b_ref, c2_ref, c2b_ref, out_ref):
    ti = jax.lax.broadcasted_iota(_i32, (56, 1), 0)
    tmask = ti < T
    for b in range(B):
        x = hT_ref[b]
        t = jnp.tanh(jnp.dot(x, a1_ref[...],
                             preferred_element_type=_f32) + a1b_ref[...])
        logits = jnp.dot(t, a2_ref[...], preferred_element_type=_f32)
        logits = jnp.where(tmask, logits, -1e30)
        e = jnp.exp(logits - jnp.max(logits, axis=0, keepdims=True))
        e = jnp.where(tmask, e, 0.0)
        w = e / jnp.sum(e, axis=0, keepdims=True)
        pooled = jnp.sum(x * w, axis=0, keepdims=True)
        mu = jnp.mean(pooled, axis=1, keepdims=True)
        var = jnp.mean((pooled - mu) ** 2, axis=1, keepdims=True)
        z = (pooled - mu) * jax.lax.rsqrt(var + 1e-5) * lng_ref[...] \
            + lnb_ref[...]
        z1 = _gelu(jnp.dot(z, c1_ref[...],
                           preferred_element_type=_f32) + c1b_ref[...])
        out_ref[pl.ds(b, 1), :] = jnp.dot(
            z1, c2_ref[...], preferred_element_type=_f32) + c2b_ref[...]


def _khead(hT, a1, a1b, a2, lng, lnb, c1, c1b, c2, c2b):
    return pl.pallas_call(
        _khead_body,
        grid=(1,),
        in_specs=[
            pl.BlockSpec((B, 56, 256), lambda i: (0, 0, 0)),
            pl.BlockSpec((256, 64), _rep),
            pl.BlockSpec((1, 64), _rep),
            pl.BlockSpec((64, 1), _rep),
            pl.BlockSpec((1, 256), _rep),
            pl.BlockSpec((1, 256), _rep),
            pl.BlockSpec((256, 256), _rep),
            pl.BlockSpec((1, 256), _rep),
            pl.BlockSpec((256, 104), _rep),
            pl.BlockSpec((1, 104), _rep),
        ],
        out_specs=pl.BlockSpec((B, 104), _rep),
        out_shape=jax.ShapeDtypeStruct((B, 104), _f32),
    )(hT, a1, a1b, a2, lng, lnb, c1, c1b, c2, c2b)


RT2 = 688


def _tmm_body(A_ref, up_ref, out_ref):
    out_ref[0] = jnp.dot(A_ref[...], up_ref[0],
                         preferred_element_type=_f32)


def _tmm(A, up):
    F = up.shape[1]
    up3 = up.reshape(B, NTP, F)
    out = pl.pallas_call(
        _tmm_body,
        grid=(NTP // RT2, B),
        in_specs=[
            pl.BlockSpec((RT2, NTP), lambda i, b: (i, 0)),
            pl.BlockSpec((1, NTP, F), lambda i, b: (b, 0, 0)),
        ],
        out_specs=pl.BlockSpec((1, RT2, F), lambda i, b: (b, i, 0)),
        out_shape=jax.ShapeDtypeStruct((B, NTP, F), _f32),
    )(A, up3)
    return out.reshape(MP, F)


def kernel(x, s_ei, t_ei, Wp, bp, g1_W, g1_b, bn1_g, bn1_b, r1_W, r1_b,
           g2_W, g2_b, bn2_g, bn2_b, g3_W, g3_b, bn3_g, bn3_b, r3_W, r3_b,
           g4_W, g4_b, bn4_g, bn4_b, a1_W, a1_b, a2_W, a2_b, ln_g, ln_b,
           c1_W, c1_b, c2_W, c2_b):
    xf = jnp.pad(x.reshape(B, NT, C), ((0, 0), (0, 2), (0, 2)))
    xf = xf.reshape(MP, 8)

    deg_s = jnp.zeros((N,), _f32).at[s_ei[1]].add(1.0)
    deg_t = jnp.zeros((NT,), _f32).at[t_ei[1]].add(1.0)
    dis_s = jax.lax.rsqrt(deg_s + 1.0)
    dis_t = jax.lax.rsqrt(deg_t + 1.0)
    pad1 = jnp.ones((2,), _f32)
    dis_s_col = jnp.tile(jnp.concatenate([jnp.tile(dis_s, T), pad1]),
                         B)[:, None]
    dis_t_col = jnp.tile(jnp.concatenate([dis_t, pad1]), B)[:, None]

    At = jnp.zeros((NTP, NTP), _f32).at[t_ei[1], t_ei[0]].add(1.0)
    goffs = (jnp.arange(T, dtype=_i32) * N)[:, None]
    sr0 = (s_ei[0][None, :] + goffs).reshape(-1)
    sc0 = (s_ei[1][None, :] + goffs).reshape(-1)
    As = jnp.zeros((NTP, NTP), _f32).at[sc0, sr0].add(1.0)

    def r2(v):
        return v[None, :]

    up1, res1 = _ka(xf, jnp.pad(Wp, ((0, 2), (0, 0))), r2(bp),
                    g1_W, r1_W, r2(r1_b), dis_s_col)
    acc1 = _tmm(As, up1)
    h1, up2 = _post_pre(acc1, up1, res1, dis_s_col, r2(g1_b),
                        r2(bn1_g) * S_BN, r2(bn1_b), g2_W, dis_s_col)
    acc2 = _tmm(As, up2)
    up3, res3 = _post_pre2(acc2, up2, h1, dis_s_col, r2(g2_b),
                           r2(bn2_g) * S_BN, r2(bn2_b), g3_W, r3_W,
                           r2(r3_b), dis_t_col)
    acc3 = _tmm(At, up3)
    h3, up4 = _post_pre(acc3, up3, res3, dis_t_col, r2(g3_b),
                        r2(bn3_g) * S_BN, r2(bn3_b), g4_W, dis_t_col)
    acc4 = _tmm(At, up4)
    hT = _kpool(acc4, up4, h3, dis_t_col, r2(g4_b),
                r2(bn4_g) * S_BN, r2(bn4_b))

    out = _khead(hT, a1_W, r2(a1_b), a2_W, r2(ln_g), r2(ln_b),
                 c1_W, r2(c1_b), jnp.pad(c2_W, ((0, 0), (0, 4))),
                 jnp.pad(r2(c2_b), ((0, 0), (0, 4))))
    return out[:, :100]
